# baseline (speedup 1.0000x reference)
import functools

import jax
import jax.numpy as jnp
from jax import lax
from jax.experimental import pallas as pl
from jax.experimental.pallas import tpu as pltpu
from jax.experimental.pallas import tpu_sc as plsc

B, L, D = 2, 2048, 768
H = 12
K = 512
NU = 256
NH = 4
SPLIT = NU // NH
QC = 128
NQ = L // QC
TOK = K + 2

NC, NS = 2, 16
NW = NC * NS
RPT = (B * K) // NW


def _body(sas_ref, hs_ref, am_ref, wq_ref, wkv_ref, wo_ref, bo_ref,
          new_ref, mask_ref, idx_ref, acc_s):
    qi = pl.program_id(1)

    @pl.when(qi == 0)
    def _init():
        acc_s[...] = jnp.zeros_like(acc_s)

    @pl.when(qi == 1)
    def _mha():
        hs = hs_ref[0]
        am = am_ref[0]
        mx = jnp.max(am)
        e = jnp.exp(am - mx)
        att = e / jnp.sum(e)
        sentences = jax.lax.dot_general(
            att, hs, (((1,), (0,)), ((), ())),
            preferred_element_type=jnp.float32)

        q_row = jax.lax.dot_general(sentences, wq_ref[...],
                                    (((1,), (0,)), ((), ())),
                                    preferred_element_type=jnp.float32)
        kvmat = jax.lax.dot_general(hs, wkv_ref[...],
                                    (((1,), (0,)), ((), ())),
                                    preferred_element_type=jnp.float32)

        kpm = am < jnp.float32(-10.0)
        scale = jnp.float32(1.0 / (768.0 ** 0.5))
        heads = []
        for h in range(NH):
            qh = q_row[:, h * SPLIT:(h + 1) * SPLIT]
            kh = kvmat[:, h * SPLIT:(h + 1) * SPLIT]
            vh = kvmat[:, NU + h * SPLIT:NU + (h + 1) * SPLIT]
            s = jax.lax.dot_general(qh, kh, (((1,), (1,)), ((), ())),
                                    preferred_element_type=jnp.float32)
            s = s * scale
            s = jnp.where(kpm, -jnp.inf, s)
            smx = jnp.max(s)
            se = jnp.exp(s - smx)
            p = se / jnp.sum(se)
            oh = jax.lax.dot_general(p, vh, (((1,), (0,)), ((), ())),
                                     preferred_element_type=jnp.float32)
            heads.append(oh)
        o = jnp.concatenate(heads, axis=1)
        new_tok = jax.lax.dot_general(o, wo_ref[...],
                                      (((1,), (0,)), ((), ())),
                                      preferred_element_type=jnp.float32)
        new_ref[0] = new_tok + bo_ref[...]

    x = sas_ref[0]
    m = x[0]
    for h in range(1, H):
        m = m + x[h]
    m = m * jnp.float32(1.0 / 12.0)
    for t in range(QC // 8):
        acc_s[...] = acc_s[...] + m[8 * t:8 * t + 8]

    @pl.when(qi == NQ - 1)
    def _select():
        acc = acc_s[...]
        a4 = acc[0:4] + acc[4:8]
        a2 = a4[0:2] + a4[2:4]
        imp = a2[0:1] + a2[1:2]

        bits = jax.lax.bitcast_convert_type(imp, jnp.int32)
        key = jnp.where(bits < 0,
                        jnp.bitwise_xor(jnp.bitwise_not(bits),
                                        jnp.int32(-2147483648)),
                        bits)

        def bit_step(i, prefix):
            b = 31 - i
            cand = jnp.where(
                b == 31,
                jnp.int32(0),
                jnp.bitwise_or(prefix, jnp.left_shift(jnp.int32(1), b)))
            cnt = jnp.sum((key >= cand).astype(jnp.int32))
            return jnp.where(cnt >= K, cand, prefix)

        kth = jax.lax.fori_loop(0, 32, bit_step, jnp.int32(-2147483648))

        sel_gt = key > kth
        eq = key == kth
        n_gt = jnp.sum(sel_gt.astype(jnp.int32))
        need_eq = K - n_gt

        lane = jax.lax.broadcasted_iota(jnp.int32, (1, L), 1)

        def excl_cumsum(v):
            s = v
            k = 1
            while k < L:
                sh = pltpu.roll(s, k, 1)
                s = s + jnp.where(lane >= k, sh, jnp.int32(0))
                k *= 2
            return s - v

        rank_eq = excl_cumsum(eq.astype(jnp.int32))
        sel = jnp.logical_or(sel_gt,
                             jnp.logical_and(eq, rank_eq < need_eq))
        rank_i = excl_cumsum(sel.astype(jnp.int32))

        kio = jax.lax.broadcasted_iota(jnp.int32, (K, L), 0)
        pmat = jnp.where(
            jnp.logical_and(jnp.broadcast_to(sel, (K, L)),
                            jnp.broadcast_to(rank_i, (K, L)) == kio),
            jnp.float32(1.0), jnp.float32(0.0))

        am = am_ref[0]
        pam = jax.lax.dot_general(
            am, pmat, (((1,), (1,)), ((), ())),
            precision=jax.lax.Precision.HIGHEST,
            preferred_element_type=jnp.float32)
        lanef = lane.astype(jnp.float32)
        idxf = jax.lax.dot_general(
            lanef, pmat, (((1,), (1,)), ((), ())),
            precision=jax.lax.Precision.HIGHEST,
            preferred_element_type=jnp.float32)
        idx_g = idxf.astype(jnp.int32) + pl.program_id(0) * L

        zero1 = jnp.zeros((1, 1), jnp.float32)
        mask_ref[0, 0] = jnp.concatenate([zero1, pam, zero1], axis=1)
        idx_ref[0] = idx_g


def _tc_call(sas, hs, am3, wq, wkv, wo, bo2):
    return pl.pallas_call(
        _body,
        grid=(B, NQ),
        in_specs=[
            pl.BlockSpec((1, H, QC, L), lambda b, q: (b, 0, q, 0)),
            pl.BlockSpec((1, L, D), lambda b, q: (b, 0, 0)),
            pl.BlockSpec((1, 1, L), lambda b, q: (b, 0, 0)),
            pl.BlockSpec((D, NU), lambda b, q: (0, 0)),
            pl.BlockSpec((D, 2 * NU), lambda b, q: (0, 0)),
            pl.BlockSpec((NU, D), lambda b, q: (0, 0)),
            pl.BlockSpec((1, D), lambda b, q: (0, 0)),
        ],
        out_specs=[
            pl.BlockSpec((1, 1, D), lambda b, q: (b, 0, 0)),
            pl.BlockSpec((1, 1, 1, TOK), lambda b, q: (b, 0, 0, 0)),
            pl.BlockSpec((1, 1, K), lambda b, q: (b, 0, 0)),
        ],
        out_shape=[
            jax.ShapeDtypeStruct((B, 1, D), jnp.float32),
            jax.ShapeDtypeStruct((B, 1, 1, TOK), jnp.float32),
            jax.ShapeDtypeStruct((B, 1, K), jnp.int32),
        ],
        scratch_shapes=[pltpu.VMEM((8, L), jnp.float32)],
        compiler_params=pltpu.CompilerParams(
            dimension_semantics=("arbitrary", "arbitrary"),
        ),
    )(sas, hs, am3, wq, wkv, wo, bo2)


def _sc_assemble(hs_flat, idx_g, new_flat):
    mesh = plsc.VectorSubcoreMesh(core_axis_name="c", subcore_axis_name="s",
                                  num_cores=NC, num_subcores=NS)

    @functools.partial(
        pl.kernel, mesh=mesh,
        out_type=jax.ShapeDtypeStruct((B * TOK, D), jnp.float32),
        scratch_types=[
            pltpu.VMEM((RPT,), jnp.int32),
            pltpu.VMEM((RPT,), jnp.int32),
            pltpu.VMEM((RPT, D), jnp.float32),
            pltpu.VMEM((2,), jnp.int32),
            pltpu.VMEM((2,), jnp.int32),
            pltpu.VMEM((2,), jnp.int32),
            pltpu.VMEM((2,), jnp.int32),
            pltpu.VMEM((2, D), jnp.float32),
            pltpu.SemaphoreType.DMA,
        ],
    )
    def k(hs_hbm, idx_hbm, new_hbm, eidx_hbm, out_hbm,
          idx_v, oidx_v, rows_v, cg_v, ng_v, cw_v, nw_v, erows_v, sem):
        wid = lax.axis_index("s") * NC + lax.axis_index("c")
        base = wid * RPT
        b = base // K
        k0 = base - b * K
        out_base = b * TOK + 1 + k0
        for c in range(RPT // 16):
            oidx_v[pl.ds(c * 16, 16)] = (
                jax.lax.broadcasted_iota(jnp.int32, (16,), 0)
                + (out_base + c * 16))
        pltpu.sync_copy(idx_hbm.at[pl.ds(base, RPT)], idx_v)
        pltpu.async_copy(hs_hbm.at[idx_v], rows_v, sem).wait()
        pltpu.async_copy(rows_v, out_hbm.at[oidx_v], sem).wait()

        @pl.when(wid == 0)
        def _edges():
            pltpu.sync_copy(eidx_hbm.at[pl.ds(0, 2)], cg_v)
            pltpu.sync_copy(eidx_hbm.at[pl.ds(8, 2)], ng_v)
            pltpu.sync_copy(eidx_hbm.at[pl.ds(16, 2)], cw_v)
            pltpu.sync_copy(eidx_hbm.at[pl.ds(24, 2)], nw_v)
            pltpu.async_copy(hs_hbm.at[cg_v], erows_v, sem).wait()
            pltpu.async_copy(erows_v, out_hbm.at[cw_v], sem).wait()
            pltpu.async_copy(new_hbm.at[ng_v], erows_v, sem).wait()
            pltpu.async_copy(erows_v, out_hbm.at[nw_v], sem).wait()

    return k(hs_flat, idx_g, new_flat, _EIDX)


_EIDX = jnp.zeros((32,), jnp.int32).at[jnp.array([0, 1, 8, 9, 16, 17, 24, 25])].set(
    jnp.array([0, L, 0, 1, 0, TOK, TOK - 1, 2 * TOK - 1], jnp.int32))


def kernel(hidden_states, attention_mask, self_attention_scores,
           Wq, Wk, Wv, Wo, bo):
    am3 = attention_mask.reshape(B, 1, L)
    wkv = jnp.concatenate([Wk, Wv], axis=1)
    bo2 = bo.reshape(1, D)
    new_tok, final_mask, idx_g = _tc_call(self_attention_scores,
                                          hidden_states, am3, Wq, wkv, Wo,
                                          bo2)
    tok_flat = _sc_assemble(hidden_states.reshape(B * L, D),
                            idx_g.reshape(B * K),
                            new_tok.reshape(B, D))
    return (tok_flat.reshape(B, TOK, D), final_mask)


# speedup vs baseline: 1.0221x; 1.0221x over previous
import functools

import jax
import jax.numpy as jnp
from jax import lax
from jax.experimental import pallas as pl
from jax.experimental.pallas import tpu as pltpu
from jax.experimental.pallas import tpu_sc as plsc

B, L, D = 2, 2048, 768
H = 12
K = 512
NU = 256
NH = 4
SPLIT = NU // NH
QC = 128
NQ = L // QC
TOK = K + 2

NC, NS = 2, 16
NW = NC * NS
RPT = (B * K) // NW


def _body(sas_ref, hs_ref, am_ref, wq_ref, wkv_ref, wo_ref, bo_ref,
          new_ref, mask_ref, idx_ref, acc_s):
    qi = pl.program_id(1)

    @pl.when(qi == 0)
    def _init():
        acc_s[...] = jnp.zeros_like(acc_s)

    @pl.when(qi == 1)
    def _mha():
        hs = hs_ref[0]
        am = am_ref[0]
        mx = jnp.max(am)
        e = jnp.exp(am - mx)
        att = e / jnp.sum(e)
        sentences = jax.lax.dot_general(
            att, hs, (((1,), (0,)), ((), ())),
            preferred_element_type=jnp.float32)

        q_row = jax.lax.dot_general(sentences, wq_ref[...],
                                    (((1,), (0,)), ((), ())),
                                    preferred_element_type=jnp.float32)
        kvmat = jax.lax.dot_general(hs, wkv_ref[...],
                                    (((1,), (0,)), ((), ())),
                                    preferred_element_type=jnp.float32)

        kpm = am < jnp.float32(-10.0)
        scale = jnp.float32(1.0 / (768.0 ** 0.5))
        heads = []
        for h in range(NH):
            qh = q_row[:, h * SPLIT:(h + 1) * SPLIT]
            kh = kvmat[:, h * SPLIT:(h + 1) * SPLIT]
            vh = kvmat[:, NU + h * SPLIT:NU + (h + 1) * SPLIT]
            s = jax.lax.dot_general(qh, kh, (((1,), (1,)), ((), ())),
                                    preferred_element_type=jnp.float32)
            s = s * scale
            s = jnp.where(kpm, -jnp.inf, s)
            smx = jnp.max(s)
            se = jnp.exp(s - smx)
            p = se / jnp.sum(se)
            oh = jax.lax.dot_general(p, vh, (((1,), (0,)), ((), ())),
                                     preferred_element_type=jnp.float32)
            heads.append(oh)
        o = jnp.concatenate(heads, axis=1)
        new_tok = jax.lax.dot_general(o, wo_ref[...],
                                      (((1,), (0,)), ((), ())),
                                      preferred_element_type=jnp.float32)
        new_ref[0] = new_tok + bo_ref[...]

    x = sas_ref[0]
    m = x[0]
    for h in range(1, H):
        m = m + x[h]
    m = m * jnp.float32(1.0 / 12.0)
    for t in range(QC // 8):
        acc_s[...] = acc_s[...] + m[8 * t:8 * t + 8]

    @pl.when(qi == NQ - 1)
    def _select():
        acc = acc_s[...]
        a4 = acc[0:4] + acc[4:8]
        a2 = a4[0:2] + a4[2:4]
        imp = a2[0:1] + a2[1:2]

        bits = jax.lax.bitcast_convert_type(imp, jnp.int32)
        key = jnp.where(bits < 0,
                        jnp.bitwise_xor(jnp.bitwise_not(bits),
                                        jnp.int32(-2147483648)),
                        bits)

        def bit_step(i, prefix):
            b = 31 - i
            cand = jnp.where(
                b == 31,
                jnp.int32(0),
                jnp.bitwise_or(prefix, jnp.left_shift(jnp.int32(1), b)))
            cnt = jnp.sum((key >= cand).astype(jnp.int32))
            return jnp.where(cnt >= K, cand, prefix)

        kth = jax.lax.fori_loop(0, 32, bit_step, jnp.int32(-2147483648))

        sel_gt = key > kth
        eq = key == kth
        n_gt = jnp.sum(sel_gt.astype(jnp.int32))
        need_eq = K - n_gt

        lane = jax.lax.broadcasted_iota(jnp.int32, (1, L), 1)

        def excl_cumsum(v):
            s = v
            k = 1
            while k < L:
                sh = pltpu.roll(s, k, 1)
                s = s + jnp.where(lane >= k, sh, jnp.int32(0))
                k *= 2
            return s - v

        rank_eq = excl_cumsum(eq.astype(jnp.int32))
        sel = jnp.logical_or(sel_gt,
                             jnp.logical_and(eq, rank_eq < need_eq))
        rank_i = excl_cumsum(sel.astype(jnp.int32))

        kio = jax.lax.broadcasted_iota(jnp.int32, (K, L), 0)
        pmat = jnp.where(
            jnp.logical_and(jnp.broadcast_to(sel, (K, L)),
                            jnp.broadcast_to(rank_i, (K, L)) == kio),
            jnp.float32(1.0), jnp.float32(0.0))

        am = am_ref[0]
        pam = jax.lax.dot_general(
            am, pmat, (((1,), (1,)), ((), ())),
            precision=jax.lax.Precision.HIGHEST,
            preferred_element_type=jnp.float32)
        lanef = lane.astype(jnp.float32)
        idxf = jax.lax.dot_general(
            lanef, pmat, (((1,), (1,)), ((), ())),
            precision=jax.lax.Precision.HIGHEST,
            preferred_element_type=jnp.float32)
        idx_g = idxf.astype(jnp.int32) + pl.program_id(0) * L

        zero1 = jnp.zeros((1, 1), jnp.float32)
        mask_ref[0, 0] = jnp.concatenate([zero1, pam, zero1], axis=1)
        idx_ref[0] = idx_g


def _tc_call(sas, hs, am3, wq, wkv, wo, bo2):
    return pl.pallas_call(
        _body,
        grid=(B, NQ),
        in_specs=[
            pl.BlockSpec((1, H, QC, L), lambda b, q: (b, 0, q, 0)),
            pl.BlockSpec((1, L, D), lambda b, q: (b, 0, 0)),
            pl.BlockSpec((1, 1, L), lambda b, q: (b, 0, 0)),
            pl.BlockSpec((D, NU), lambda b, q: (0, 0)),
            pl.BlockSpec((D, 2 * NU), lambda b, q: (0, 0)),
            pl.BlockSpec((NU, D), lambda b, q: (0, 0)),
            pl.BlockSpec((1, D), lambda b, q: (0, 0)),
        ],
        out_specs=[
            pl.BlockSpec((1, 1, D), lambda b, q: (b, 0, 0)),
            pl.BlockSpec((1, 1, 1, TOK), lambda b, q: (b, 0, 0, 0)),
            pl.BlockSpec((1, 1, K), lambda b, q: (b, 0, 0)),
        ],
        out_shape=[
            jax.ShapeDtypeStruct((B, 1, D), jnp.float32),
            jax.ShapeDtypeStruct((B, 1, 1, TOK), jnp.float32),
            jax.ShapeDtypeStruct((B, 1, K), jnp.int32),
        ],
        scratch_shapes=[pltpu.VMEM((8, L), jnp.float32)],
        compiler_params=pltpu.CompilerParams(
            dimension_semantics=("arbitrary", "arbitrary"),
        ),
    )(sas, hs, am3, wq, wkv, wo, bo2)


def _sc_gather(hs_flat, idx_g):
    mesh = plsc.VectorSubcoreMesh(core_axis_name="c", subcore_axis_name="s",
                                  num_cores=NC, num_subcores=NS)

    @functools.partial(
        pl.kernel, mesh=mesh,
        out_type=jax.ShapeDtypeStruct((B * K, D), jnp.float32),
        scratch_types=[
            pltpu.VMEM((RPT,), jnp.int32),
            pltpu.VMEM((RPT, D), jnp.float32),
            pltpu.SemaphoreType.DMA,
        ],
    )
    def k(hs_hbm, idx_hbm, out_hbm, idx_v, rows_v, sem):
        wid = lax.axis_index("s") * NC + lax.axis_index("c")
        base = wid * RPT
        pltpu.sync_copy(idx_hbm.at[pl.ds(base, RPT)], idx_v)
        pltpu.async_copy(hs_hbm.at[idx_v], rows_v, sem).wait()
        pltpu.sync_copy(rows_v, out_hbm.at[pl.ds(base, RPT)])

    return k(hs_flat, idx_g)


def kernel(hidden_states, attention_mask, self_attention_scores,
           Wq, Wk, Wv, Wo, bo):
    am3 = attention_mask.reshape(B, 1, L)
    wkv = jnp.concatenate([Wk, Wv], axis=1)
    bo2 = bo.reshape(1, D)
    new_tok, final_mask, idx_g = _tc_call(self_attention_scores,
                                          hidden_states, am3, Wq, wkv, Wo,
                                          bo2)
    preserved = _sc_gather(hidden_states.reshape(B * L, D),
                           idx_g.reshape(B * K))
    final_token = jnp.concatenate(
        [hidden_states[:, :1, :], preserved.reshape(B, K, D), new_tok],
        axis=1)
    return (final_token, final_mask)
